# baseline (speedup 1.0000x reference)
import functools

import jax
import jax.numpy as jnp
from jax import lax
from jax.experimental import pallas as pl
from jax.experimental.pallas import tpu as pltpu
from jax.experimental.pallas import tpu_sc as plsc

BATCH = 16384
HIST = 50
EMBED = 32
NUM_CORES = 2
NUM_SUBCORES = 16
NW = NUM_CORES * NUM_SUBCORES
BSTRIPE = BATCH // NW
HALF = BSTRIPE // 2
NGROUP = HALF // 16
TROWS = 250000
OBLK_STRIDE = HALF + 8


def _sc_gather(table2, idx_t):
    mesh = plsc.VectorSubcoreMesh(core_axis_name="c", subcore_axis_name="s")

    @functools.partial(
        pl.kernel,
        out_type=jax.ShapeDtypeStruct((HIST, EMBED, BATCH), jnp.float32),
        mesh=mesh,
        scratch_types=(
            [pltpu.VMEM((HALF,), jnp.int32) for _ in range(2)]
            + [pltpu.VMEM((HALF,), jnp.int32) for _ in range(2)]
            + [pltpu.VMEM((HALF, 128), jnp.float32) for _ in range(2)]
            + [pltpu.VMEM((EMBED, OBLK_STRIDE), jnp.float32) for _ in range(2)]
            + [pltpu.SemaphoreType.DMA for _ in range(6)]
        ),
        compiler_params=pltpu.CompilerParams(
            use_tc_tiling_on_sc=True, needs_layout_passes=False,
            disable_bounds_checks=True),
    )
    def k(table_hbm, idx_hbm, out_hbm, *scratch):
        idx_v = scratch[0:2]
        q_v = scratch[2:4]
        rows_v = scratch[4:6]
        oblk_v = scratch[6:8]
        isem = scratch[8:10]
        gsem = scratch[10:12]
        ssem = scratch[12:14]
        wid = lax.axis_index("s") * NUM_CORES + lax.axis_index("c")
        b0 = wid * BSTRIPE

        iota = lax.iota(jnp.int32, 16)

        def idx_copy(h, s):
            return pltpu.make_async_copy(
                idx_hbm.at[h, pl.ds(b0 + s * HALF, HALF)], idx_v[s], isem[s])

        def gather_copy(s):
            return pltpu.make_async_copy(
                table_hbm.at[q_v[s]], rows_v[s], gsem[s])

        def out_copy(h, s):
            return pltpu.make_async_copy(
                oblk_v[s].at[pl.ds(0, EMBED), pl.ds(0, HALF)],
                out_hbm.at[h, pl.ds(0, EMBED), pl.ds(b0 + s * HALF, HALF)],
                ssem[s])

        def q_compute(s):
            @pl.loop(0, NGROUP)
            def _(g):
                off = g * 16
                q_v[s][pl.ds(off, 16)] = idx_v[s][pl.ds(off, 16)] >> 2

        def compact(s):
            @plsc.parallel_loop(0, NGROUP)
            def _(g):
                off = g * 16
                roffv = (idx_v[s][pl.ds(off, 16)] & 3) * 32
                for l in range(16):
                    n = off + l
                    roff = roffv[l]
                    nvec = jnp.full((16,), n, jnp.int32)
                    plsc.store_scatter(
                        oblk_v[s], [iota, nvec],
                        rows_v[s][n, pl.ds(roff, 16)])
                    plsc.store_scatter(
                        oblk_v[s], [iota + 16, nvec],
                        rows_v[s][n, pl.ds(roff + 16, 16)])

        def stage_front(h, s):
            idx_copy(h, s).wait()
            q_compute(s)

            @pl.when(h > 0)
            def _():
                out_copy(h - 1, s).wait()

            gather_copy(s).start()

        def stage_back(h, s):
            gather_copy(s).wait()
            compact(s)
            out_copy(h, s).start()

            @pl.when(h < HIST - 1)
            def _():
                idx_copy(h + 1, s).start()

        idx_copy(0, 0).start()
        idx_copy(0, 1).start()
        stage_front(0, 0)

        @pl.loop(0, HIST - 1)
        def _(h):
            stage_front(h, 1)
            stage_back(h, 0)
            stage_front(h + 1, 0)
            stage_back(h, 1)

        stage_front(HIST - 1, 1)
        stage_back(HIST - 1, 0)
        stage_back(HIST - 1, 1)

        out_copy(HIST - 1, 0).wait()
        out_copy(HIST - 1, 1).wait()

    return k(table2, idx_t)


@jax.jit
def kernel(inputs, embedding):
    table2 = jnp.reshape(embedding, (TROWS, 128))
    idx_t = inputs.T
    out_p = _sc_gather(table2, idx_t)
    return jnp.transpose(out_p, (2, 0, 1))


# speedup vs baseline: 1.0253x; 1.0253x over previous
import functools

import jax
import jax.numpy as jnp
from jax import lax
from jax.experimental import pallas as pl
from jax.experimental.pallas import tpu as pltpu
from jax.experimental.pallas import tpu_sc as plsc

BATCH = 16384
HIST = 50
EMBED = 32
NUM_CORES = 2
NUM_SUBCORES = 16
NW = NUM_CORES * NUM_SUBCORES
BSTRIPE = BATCH // NW
HALF = BSTRIPE // 2
NGROUP = HALF // 16
TROWS = 250000


def _sc_gather(table2, idx_t):
    mesh = plsc.VectorSubcoreMesh(core_axis_name="c", subcore_axis_name="s")

    @functools.partial(
        pl.kernel,
        out_type=jax.ShapeDtypeStruct((HIST, EMBED, BATCH), jnp.float32),
        mesh=mesh,
        scratch_types=(
            [pltpu.VMEM((HALF,), jnp.int32) for _ in range(2)]
            + [pltpu.VMEM((HALF,), jnp.int32) for _ in range(2)]
            + [pltpu.VMEM((HALF, 128), jnp.float32) for _ in range(2)]
            + [pltpu.VMEM((EMBED, HALF), jnp.float32) for _ in range(2)]
            + [pltpu.SemaphoreType.DMA for _ in range(6)]
        ),
        compiler_params=pltpu.CompilerParams(
            use_tc_tiling_on_sc=True, needs_layout_passes=False,
            disable_bounds_checks=True),
    )
    def k(table_hbm, idx_hbm, out_hbm, *scratch):
        idx_v = scratch[0:2]
        q_v = scratch[2:4]
        rows_v = scratch[4:6]
        oblk_v = scratch[6:8]
        isem = scratch[8:10]
        gsem = scratch[10:12]
        ssem = scratch[12:14]
        wid = lax.axis_index("s") * NUM_CORES + lax.axis_index("c")
        b0 = wid * BSTRIPE

        iota = lax.iota(jnp.int32, 16)

        def idx_copy(h, s):
            return pltpu.make_async_copy(
                idx_hbm.at[h, pl.ds(b0 + s * HALF, HALF)], idx_v[s], isem[s])

        def gather_copy(s):
            return pltpu.make_async_copy(
                table_hbm.at[q_v[s]], rows_v[s], gsem[s])

        def out_copy(h, s):
            return pltpu.make_async_copy(
                oblk_v[s],
                out_hbm.at[h, pl.ds(0, EMBED), pl.ds(b0 + s * HALF, HALF)],
                ssem[s])

        def q_compute(s):
            @pl.loop(0, NGROUP)
            def _(g):
                off = g * 16
                q_v[s][pl.ds(off, 16)] = idx_v[s][pl.ds(off, 16)] >> 2

        def compact(s):
            @plsc.parallel_loop(0, NGROUP)
            def _(g):
                off = g * 16
                idxg = idx_v[s][pl.ds(off, 16)]
                roff = (idxg & 3) << 5
                srow = off + iota
                for c in range(EMBED):
                    oblk_v[s][c, pl.ds(off, 16)] = plsc.load_gather(
                        rows_v[s], [srow, roff + c])

        def stage_front(h, s):
            idx_copy(h, s).wait()
            q_compute(s)

            @pl.when(h > 0)
            def _():
                out_copy(h - 1, s).wait()

            gather_copy(s).start()

        def stage_back(h, s):
            gather_copy(s).wait()
            compact(s)
            out_copy(h, s).start()

            @pl.when(h < HIST - 1)
            def _():
                idx_copy(h + 1, s).start()

        idx_copy(0, 0).start()
        idx_copy(0, 1).start()
        stage_front(0, 0)

        @pl.loop(0, HIST - 1)
        def _(h):
            stage_front(h, 1)
            stage_back(h, 0)
            stage_front(h + 1, 0)
            stage_back(h, 1)

        stage_front(HIST - 1, 1)
        stage_back(HIST - 1, 0)
        stage_back(HIST - 1, 1)

        out_copy(HIST - 1, 0).wait()
        out_copy(HIST - 1, 1).wait()

    return k(table2, idx_t)


@jax.jit
def kernel(inputs, embedding):
    table2 = jnp.reshape(embedding, (TROWS, 128))
    idx_t = inputs.T
    out_p = _sc_gather(table2, idx_t)
    return jnp.transpose(out_p, (2, 0, 1))
